# Initial kernel scaffold; baseline (speedup 1.0000x reference)
import jax
import jax.numpy as jnp
from jax.experimental import pallas as pl


def kernel(u, states, adj_list, adj_list_mask, lut, w_in, W, b):
    raise NotImplementedError("write your pallas kernel here")


# speedup vs baseline: 4.2903x; 4.2903x over previous
import functools

import jax
import jax.numpy as jnp
from jax import lax
from jax.experimental import pallas as pl
from jax.experimental.pallas import tpu as pltpu
from jax.experimental.pallas import tpu_sc as plsc

N_NODES = 50000
N_INPUT = 2048
N_RES = N_NODES - N_INPUT
K = 8
N_OUT = 16
B = 64
LUT_W = 256

NUM_WORKERS = 32
NB = 16
CHUNK = 1504
assert (NUM_WORKERS - 1) * CHUNK + 1328 == N_RES
assert CHUNK % NB == 0 and 1328 % NB == 0


def _sc_gather_lut(states_t, adj_flat, lut):
    mesh = plsc.VectorSubcoreMesh(core_axis_name="c", subcore_axis_name="s")

    @functools.partial(
        pl.kernel,
        mesh=mesh,
        compiler_params=pltpu.CompilerParams(
            needs_layout_passes=False, use_tc_tiling_on_sc=False),
        out_type=jax.ShapeDtypeStruct((N_RES, B), jnp.float32),
        scratch_types=[
            pltpu.VMEM((NB * K,), jnp.int32),
            pltpu.VMEM((NB * K, B), jnp.float32),
            pltpu.VMEM((NB, LUT_W), jnp.float32),
            pltpu.VMEM((NB, B), jnp.float32),
            pltpu.SemaphoreType.DMA,
        ],
    )
    def k(st_hbm, adj_hbm, lut_hbm, out_hbm, aidx_v, neigh_v, lutb_v, outb_v, sem):
        wid = lax.axis_index("s") * 2 + lax.axis_index("c")
        start = wid * CHUNK
        nblk = jnp.where(wid == NUM_WORKERS - 1, 1328 // NB, CHUNK // NB)

        def body(blk, carry):
            g = start + blk * NB
            pltpu.sync_copy(adj_hbm.at[pl.ds((g + N_INPUT) * K, NB * K)], aidx_v)
            gcp = pltpu.async_copy(st_hbm.at[aidx_v], neigh_v, sem)
            pltpu.sync_copy(lut_hbm.at[pl.ds(g + N_INPUT, NB)], lutb_v)
            gcp.wait()
            for i in range(NB):
                row = jnp.full((16,), i, jnp.int32)
                for bv in range(B // 16):
                    sl = pl.ds(bv * 16, 16)
                    acc = neigh_v[i * K, sl]
                    for kk in range(1, K):
                        acc = acc + neigh_v[i * K + kk, sl] * float(2 ** kk)
                    idx = acc.astype(jnp.int32)
                    outb_v[i, sl] = plsc.load_gather(lutb_v, [row, idx])
            pltpu.sync_copy(outb_v, out_hbm.at[pl.ds(g, NB)])
            return carry

        lax.fori_loop(0, nblk, body, 0)

    return k(states_t, adj_flat, lut)


def _tc_readout(w_t, res_t, b2):
    def body(w_ref, r_ref, b_ref, o_ref):
        logits = lax.dot_general(
            w_ref[...], r_ref[...], (((1,), (0,)), ((), ())),
            preferred_element_type=jnp.float32)
        o_ref[...] = jax.nn.sigmoid(logits + b_ref[...])

    return pl.pallas_call(
        body,
        out_shape=jax.ShapeDtypeStruct((N_OUT, B), jnp.float32),
    )(w_t, res_t, b2)


def kernel(u, states, adj_list, adj_list_mask, lut, w_in, W, b):
    del u, adj_list_mask, w_in
    states_t = states.T
    adj_flat = adj_list.reshape(-1)
    res_t = _sc_gather_lut(states_t, adj_flat, lut)
    out_t = _tc_readout(W.T, res_t, b.reshape(N_OUT, 1))
    return out_t.T
